# Initial kernel scaffold; baseline (speedup 1.0000x reference)
import jax
import jax.numpy as jnp
from jax.experimental import pallas as pl


def kernel(embeddings, memory_bank, output_size):
    raise NotImplementedError("write your pallas kernel here")


# speedup vs baseline: 9.7424x; 9.7424x over previous
import numpy as np
import jax
import jax.numpy as jnp
from jax import lax
from jax.experimental import pallas as pl

_H = 32
_W = 32
_B = 32
_C = 256
_N = 128
_WIN = 5
_HB = _H + _WIN - 1
_WB = _W + _WIN - 1
_WG = 8
_NG = _W // _WG
_JW = _WG + _WIN - 1

_SIGMA = 4.0
_KS = 2 * int(4.0 * _SIGMA + 0.5) + 1
_OUT = 256


def _build_post_matrix():
    scale = _OUT / _H
    o = np.arange(_OUT, dtype=np.float64)
    x = (o + 0.5) / scale - 0.5
    i = np.arange(_H, dtype=np.float64)
    w = np.maximum(0.0, 1.0 - np.abs(x[:, None] - i[None, :]))
    w = w / w.sum(axis=1, keepdims=True)
    xs = np.arange(_KS, dtype=np.float64) - (_KS - 1) / 2.0
    k = np.exp(-0.5 * (xs / _SIGMA) ** 2)
    k = k / k.sum()
    pad = _KS // 2
    idx = np.arange(-pad, _OUT + pad)
    idx = np.where(idx < 0, -idx, idx)
    idx = np.where(idx > _OUT - 1, 2 * (_OUT - 1) - idx, idx)
    P = np.zeros((_OUT + 2 * pad, _OUT))
    P[np.arange(_OUT + 2 * pad), idx] = 1.0
    K = np.zeros((_OUT, _OUT + 2 * pad))
    for r in range(_OUT):
        K[r, r:r + _KS] = k
    return ((K @ P) @ w).astype(np.float32)


_M_CONST = _build_post_matrix()


def _dist_kernel(emb_ref, bank_ref, s_ref):
    d1 = pl.program_id(1)
    wl = lax.broadcasted_iota(jnp.int32, (_N, _WG * _B), 1) // _B
    for g in range(_NG):
        gs = g * _WG
        xg = emb_ref[0, gs:gs + _WG].reshape(_WG * _B, _C)
        yg = bank_ref[0, gs:gs + _JW].reshape(_JW * _N, _C)
        yn = jnp.sum(yg * yg, axis=1, keepdims=True)
        dot = lax.dot_general(yg, xg, (((1,), (1,)), ((), ())),
                              preferred_element_type=jnp.float32)
        acc = jnp.full((_N, _WG * _B), jnp.inf, jnp.float32)
        for jj in range(_JW):
            cand = yn[jj * _N:(jj + 1) * _N] - 2.0 * dot[jj * _N:(jj + 1) * _N]
            mask = (wl <= jj) & (jj <= wl + _WIN - 1)
            acc = jnp.where(mask, jnp.minimum(acc, cand), acc)
        best = jnp.min(acc, axis=0)
        sl = slice(gs * _B, (gs + _WG) * _B)

        @pl.when(d1 == 0)
        def _():
            s_ref[0, 0, sl] = best

        @pl.when(d1 > 0)
        def _():
            s_ref[0, 0, sl] = jnp.minimum(s_ref[0, 0, sl], best)

        @pl.when(d1 == _WIN - 1)
        def _():
            xsq = xg * xg
            xn = lax.dot_general(jnp.ones((1, _C), jnp.float32), xsq,
                                 (((1,), (1,)), ((), ())),
                                 preferred_element_type=jnp.float32)[0]
            s_ref[0, 0, sl] = jnp.sqrt(
                jnp.maximum(s_ref[0, 0, sl] + xn, 0.0))


def _post_kernel(s_ref, sbw_ref, m_ref, mt_ref, out_ref, img_ref):
    s = s_ref[...].reshape(_H, _W * _B)
    v = jnp.max(s, axis=0, keepdims=True)
    k = _W * _B // 2
    while k >= _B:
        v = jnp.maximum(v[:, :k], v[:, k:2 * k])
        k //= 2
    img_ref[...] = v

    sbw = sbw_ref[...]
    m = m_ref[...]
    mt = mt_ref[...]
    t1 = lax.dot_general(m, sbw, (((1,), (0,)), ((), ())),
                         preferred_element_type=jnp.float32)
    for b in range(_B):
        out_ref[b] = lax.dot_general(
            t1[:, b * _W:(b + 1) * _W], mt, (((1,), (0,)), ((), ())),
            preferred_element_type=jnp.float32)


def kernel(embeddings, memory_bank, output_size):
    del output_size
    s = pl.pallas_call(
        _dist_kernel,
        grid=(_H, _WIN),
        in_specs=[
            pl.BlockSpec((1, _W, _B, _C), lambda h, d: (h, 0, 0, 0)),
            pl.BlockSpec((1, _WB, _N, _C), lambda h, d: (h + d, 0, 0, 0)),
        ],
        out_specs=pl.BlockSpec((1, 1, _W * _B), lambda h, d: (h, 0, 0)),
        out_shape=jax.ShapeDtypeStruct((_H, 1, _W * _B), jnp.float32),
    )(embeddings, memory_bank)

    s_bw = jnp.transpose(s.reshape(_H, _W, _B), (0, 2, 1)).reshape(_H, _B * _W)
    m = jnp.asarray(_M_CONST)
    mt = jnp.asarray(_M_CONST.T)
    out, img = pl.pallas_call(
        _post_kernel,
        out_shape=[
            jax.ShapeDtypeStruct((_B, _OUT, _OUT), jnp.float32),
            jax.ShapeDtypeStruct((1, _B), jnp.float32),
        ],
    )(s, s_bw, m, mt)
    return (img.reshape(_B), out[:, None, :, :])
